# baseline (speedup 1.0000x reference)
import jax
import jax.numpy as jnp
from jax.experimental import pallas as pl
from jax.experimental.pallas import tpu as pltpu

P = 49
R = 256
C = 256
CM = 128
TIMES = 2
NEA = 3
FAR = 3

_TAPS = [(ky - 1, kx - 1) for ky in range(3) for kx in range(3)]


def _relation_kernel(p_ref, pt_ref, x0_ref, w1_ref, w2_ref, b1_ref, b2_ref,
                     out_ref, xb_ref, a1_ref, s_ref, sem):
    n = pl.program_id(0)
    cp = pltpu.make_async_copy(x0_ref.at[n], out_ref.at[0], sem)
    cp.start()
    p = p_ref[0]
    pt = pt_ref[0]
    x1c, y1c, x2c, y2c = p[:, 0:1], p[:, 1:2], p[:, 2:3], p[:, 3:4]
    x1r, y1r, x2r, y2r = pt[0:1, :], pt[1:2, :], pt[2:3, :], pt[3:4, :]
    wc = x2c - x1c + 1.0
    hc = y2c - y1c + 1.0
    wr = x2r - x1r + 1.0
    hr = y2r - y1r + 1.0
    cxc = x1c + 0.5 * wc
    cyc = y1c + 0.5 * hc
    cxr = x1r + 0.5 * wr
    cyr = y1r + 0.5 * hr
    sq = (cxc - cxr) ** 2 + (cyc - cyr) ** 2
    pos = sq > 0
    dist = jnp.where(pos, jnp.sqrt(jnp.where(pos, sq, 1.0)), 0.0)
    iw = jnp.maximum(jnp.minimum(x2r, x2c) - jnp.maximum(x1r, x1c) + 1.0, 0.0)
    ih = jnp.maximum(jnp.minimum(y2r, y2c) - jnp.maximum(y1r, y1c) + 1.0, 0.0)
    inter = iw * ih
    union = hc * wc + hr * wr - inter
    ov = inter / union
    dmax = jnp.max(dist)
    dist_nea = jnp.where(ov != 0.0, dist, 0.0)
    dist_far = jnp.where(ov == 0.0, dist, dmax)

    cols = jax.lax.broadcasted_iota(jnp.int32, (R, R), 1)
    ids = []
    ws = []
    a = dist_nea
    for _ in range(NEA):
        m = jnp.max(a, axis=1, keepdims=True)
        idx = jnp.min(jnp.where(a == m, cols, R), axis=1, keepdims=True)
        ids.append(idx)
        ws.append(m)
        a = jnp.where(cols == idx, -jnp.inf, a)
    b = dist_far
    for _ in range(FAR):
        m = jnp.min(b, axis=1, keepdims=True)
        idx = jnp.min(jnp.where(b == m, cols, R), axis=1, keepdims=True)
        ids.append(idx)
        ws.append(m)
        b = jnp.where(cols == idx, jnp.inf, b)

    w6 = jnp.concatenate(ws, axis=1) * 0.01
    mx = jnp.max(w6, axis=1, keepdims=True)
    e = jnp.exp(w6 - mx)
    w6 = e / jnp.sum(e, axis=1, keepdims=True)

    s = jnp.zeros((R, R), jnp.float32)
    for k in range(6):
        s = s + jnp.where(cols == ids[k], w6[:, k:k + 1], 0.0)
    s_ref[...] = s.astype(jnp.bfloat16)

    cp.wait()

    for _t in range(TIMES):
        xb_ref[...] = out_ref[0].astype(jnp.bfloat16)

        def conv1_body(pp, carry):
            acc = jnp.zeros((R, CM), jnp.float32) + b1_ref[...]
            y = pp // 7
            x = pp - 7 * y
            for t, (dy, dx) in enumerate(_TAPS):
                off = dy * 7 + dx
                src = jnp.clip(pp + off, 0, P - 1)
                valid = ((y + dy >= 0) & (y + dy < 7)
                         & (x + dx >= 0) & (x + dx < 7))
                m = jnp.where(valid, 1.0, 0.0)
                xs = xb_ref[pl.ds(src * R, R), :]
                acc = acc + m * jnp.dot(xs, w1_ref[t],
                                        preferred_element_type=jnp.float32)
            a1_ref[pl.ds(pp * R, R), :] = acc.astype(jnp.bfloat16)
            return carry

        jax.lax.fori_loop(0, P, conv1_body, 0)

        def conv2_body(pp, carry):
            acc = jnp.zeros((R, C), jnp.float32) + b2_ref[...]
            y = pp // 7
            x = pp - 7 * y
            for t, (dy, dx) in enumerate(_TAPS):
                off = dy * 7 + dx
                src = jnp.clip(pp + off, 0, P - 1)
                valid = ((y + dy >= 0) & (y + dy < 7)
                         & (x + dx >= 0) & (x + dx < 7))
                m = jnp.where(valid, 1.0, 0.0)
                a1s = a1_ref[pl.ds(src * R, R), :]
                acc = acc + m * jnp.dot(a1s, w2_ref[t],
                                        preferred_element_type=jnp.float32)
            base = pp * R
            rel = jnp.dot(s_ref[...], acc.astype(jnp.bfloat16),
                          preferred_element_type=jnp.float32)
            out_ref[0, pl.ds(base, R), :] = out_ref[0, pl.ds(base, R), :] + rel
            return carry

        jax.lax.fori_loop(0, P, conv2_body, 0)


def kernel(proposals, pooled_feat, W1, b1, W2, b2):
    n_img = proposals.shape[0]
    pf = pooled_feat.reshape(n_img, R, C, 7, 7)
    x0 = pf.transpose(0, 3, 4, 1, 2).reshape(n_img, P * R, C)
    w1t = jnp.pad(W1.transpose(2, 3, 1, 0).reshape(9, C, 81),
                  ((0, 0), (0, 0), (0, CM - 81))).astype(jnp.bfloat16)
    w2t = jnp.pad(W2.transpose(2, 3, 1, 0).reshape(9, 81, C),
                  ((0, 0), (0, CM - 81), (0, 0))).astype(jnp.bfloat16)
    b1p = jnp.pad(b1, (0, CM - 81)).reshape(1, CM)
    b2p = b2.reshape(1, C)
    pt = proposals.transpose(0, 2, 1)

    out = pl.pallas_call(
        _relation_kernel,
        grid=(n_img,),
        in_specs=[
            pl.BlockSpec((1, R, 4), lambda n: (n, 0, 0)),
            pl.BlockSpec((1, 4, R), lambda n: (n, 0, 0)),
            pl.BlockSpec(memory_space=pl.ANY),
            pl.BlockSpec((9, C, CM), lambda n: (0, 0, 0)),
            pl.BlockSpec((9, CM, C), lambda n: (0, 0, 0)),
            pl.BlockSpec((1, CM), lambda n: (0, 0)),
            pl.BlockSpec((1, C), lambda n: (0, 0)),
        ],
        out_specs=pl.BlockSpec((1, P * R, C), lambda n: (n, 0, 0)),
        out_shape=jax.ShapeDtypeStruct((n_img, P * R, C), jnp.float32),
        scratch_shapes=[
            pltpu.VMEM((P * R, C), jnp.bfloat16),
            pltpu.VMEM((P * R, CM), jnp.bfloat16),
            pltpu.VMEM((R, R), jnp.bfloat16),
            pltpu.SemaphoreType.DMA,
        ],
    )(proposals, pt, x0, w1t, w2t, b1p, b2p)

    y = out.reshape(n_img, 7, 7, R, C).transpose(0, 3, 4, 1, 2)
    return y.reshape(n_img * R, C, 7, 7)


# speedup vs baseline: 1.2314x; 1.2314x over previous
import jax
import jax.numpy as jnp
from jax.experimental import pallas as pl
from jax.experimental.pallas import tpu as pltpu

P = 49
R = 256
C = 256
CM = 128
TIMES = 2
NEA = 3
FAR = 3

_TAPS = [(ky - 1, kx - 1) for ky in range(3) for kx in range(3)]


def _taps():
    res = []
    for t, (dy, dx) in enumerate(_TAPS):
        y0, y1 = max(0, -dy), 7 - max(0, dy)
        x0, x1 = max(0, -dx), 7 - max(0, dx)
        spans = []
        if dx == 0:
            o = y0 * 7 * R
            spans.append((o, o + dy * 7 * R, (y1 - y0) * 7 * R))
        else:
            for yy in range(y0, y1):
                o = (yy * 7 + x0) * R
                spans.append((o, o + (dy * 7 + dx) * R, (x1 - x0) * R))
        res.append((t, spans))
    return res


_TAP_SPANS = _taps()


def _relation_kernel(p_ref, pt_ref, x0_ref, w1_ref, w2_ref, b1_ref, b2_ref,
                     out_ref, xb_ref, a1f_ref, a1b_ref, sa1_ref, s_ref, sem):
    n = pl.program_id(0)
    cp = pltpu.make_async_copy(x0_ref.at[n], out_ref.at[0], sem)
    cp.start()
    p = p_ref[0]
    pt = pt_ref[0]
    x1c, y1c, x2c, y2c = p[:, 0:1], p[:, 1:2], p[:, 2:3], p[:, 3:4]
    x1r, y1r, x2r, y2r = pt[0:1, :], pt[1:2, :], pt[2:3, :], pt[3:4, :]
    wc = x2c - x1c + 1.0
    hc = y2c - y1c + 1.0
    wr = x2r - x1r + 1.0
    hr = y2r - y1r + 1.0
    cxc = x1c + 0.5 * wc
    cyc = y1c + 0.5 * hc
    cxr = x1r + 0.5 * wr
    cyr = y1r + 0.5 * hr
    sq = (cxc - cxr) ** 2 + (cyc - cyr) ** 2
    pos = sq > 0
    dist = jnp.where(pos, jnp.sqrt(jnp.where(pos, sq, 1.0)), 0.0)
    iw = jnp.maximum(jnp.minimum(x2r, x2c) - jnp.maximum(x1r, x1c) + 1.0, 0.0)
    ih = jnp.maximum(jnp.minimum(y2r, y2c) - jnp.maximum(y1r, y1c) + 1.0, 0.0)
    inter = iw * ih
    union = hc * wc + hr * wr - inter
    ov = inter / union
    dmax = jnp.max(dist)
    dist_nea = jnp.where(ov != 0.0, dist, 0.0)
    dist_far = jnp.where(ov == 0.0, dist, dmax)

    cols = jax.lax.broadcasted_iota(jnp.int32, (R, R), 1)
    ids = []
    ws = []
    a = dist_nea
    for _ in range(NEA):
        m = jnp.max(a, axis=1, keepdims=True)
        idx = jnp.min(jnp.where(a == m, cols, R), axis=1, keepdims=True)
        ids.append(idx)
        ws.append(m)
        a = jnp.where(cols == idx, -jnp.inf, a)
    b = dist_far
    for _ in range(FAR):
        m = jnp.min(b, axis=1, keepdims=True)
        idx = jnp.min(jnp.where(b == m, cols, R), axis=1, keepdims=True)
        ids.append(idx)
        ws.append(m)
        b = jnp.where(cols == idx, jnp.inf, b)

    w6 = jnp.concatenate(ws, axis=1) * 0.01
    mx = jnp.max(w6, axis=1, keepdims=True)
    e = jnp.exp(w6 - mx)
    w6 = e / jnp.sum(e, axis=1, keepdims=True)

    s = jnp.zeros((R, R), jnp.float32)
    for k in range(6):
        s = s + jnp.where(cols == ids[k], w6[:, k:k + 1], 0.0)
    s_ref[...] = s.astype(jnp.bfloat16)

    cp.wait()

    YR = 7 * R

    def _conv(read, rmw, w_ref):
        for dy in (-1, 0, 1):
            t = (dy + 1) * 3 + 1
            y0, y1 = max(0, -dy), 7 - max(0, dy)
            o, n = y0 * YR, (y1 - y0) * YR
            rmw(o, n, jnp.dot(read(o + dy * YR, n), w_ref[t],
                              preferred_element_type=jnp.float32))
        for dy in (-1, 0, 1):
            y0, y1 = max(0, -dy), 7 - max(0, dy)

            def body(yv, c, dy=dy):
                base = yv * YR
                for dx in (-1, 1):
                    t = (dy + 1) * 3 + (dx + 1)
                    o = base + (R if dx == -1 else 0)
                    s = o + dy * YR + dx * R
                    n = 6 * R
                    rmw(o, n, jnp.dot(read(s, n), w_ref[t],
                                      preferred_element_type=jnp.float32))
                return c

            jax.lax.fori_loop(y0, y1, body, 0)

    for _t in range(TIMES):
        xb_ref[...] = out_ref[0].astype(jnp.bfloat16)

        a1f_ref[...] = jnp.zeros((P * R, CM), jnp.float32) + b1_ref[...]

        def rmw_a1(o, n, d):
            a1f_ref[pl.ds(o, n), :] = a1f_ref[pl.ds(o, n), :] + d

        _conv(lambda o, n: xb_ref[pl.ds(o, n), :], rmw_a1, w1_ref)
        a1b_ref[...] = a1f_ref[...].astype(jnp.bfloat16)

        def sa_body(pp, c):
            sl = pl.ds(pp * R, R)
            sa1_ref[sl, :] = jnp.dot(
                s_ref[...], a1b_ref[sl, :],
                preferred_element_type=jnp.float32).astype(jnp.bfloat16)
            return c

        jax.lax.fori_loop(0, P, sa_body, 0)

        out_ref[0] = out_ref[0] + (jnp.zeros((P * R, C), jnp.float32)
                                   + b2_ref[...])

        def rmw_out(o, n, d):
            out_ref[0, pl.ds(o, n), :] = out_ref[0, pl.ds(o, n), :] + d

        _conv(lambda o, n: sa1_ref[pl.ds(o, n), :], rmw_out, w2_ref)


def kernel(proposals, pooled_feat, W1, b1, W2, b2):
    n_img = proposals.shape[0]
    pf = pooled_feat.reshape(n_img, R, C, 7, 7)
    x0 = pf.transpose(0, 3, 4, 1, 2).reshape(n_img, P * R, C)
    w1t = jnp.pad(W1.transpose(2, 3, 1, 0).reshape(9, C, 81),
                  ((0, 0), (0, 0), (0, CM - 81))).astype(jnp.bfloat16)
    w2t = jnp.pad(W2.transpose(2, 3, 1, 0).reshape(9, 81, C),
                  ((0, 0), (0, CM - 81), (0, 0))).astype(jnp.bfloat16)
    b1p = jnp.pad(b1, (0, CM - 81)).reshape(1, CM)
    b2p = b2.reshape(1, C)
    pt = proposals.transpose(0, 2, 1)

    out = pl.pallas_call(
        _relation_kernel,
        grid=(n_img,),
        in_specs=[
            pl.BlockSpec((1, R, 4), lambda n: (n, 0, 0)),
            pl.BlockSpec((1, 4, R), lambda n: (n, 0, 0)),
            pl.BlockSpec(memory_space=pl.ANY),
            pl.BlockSpec((9, C, CM), lambda n: (0, 0, 0)),
            pl.BlockSpec((9, CM, C), lambda n: (0, 0, 0)),
            pl.BlockSpec((1, CM), lambda n: (0, 0)),
            pl.BlockSpec((1, C), lambda n: (0, 0)),
        ],
        out_specs=pl.BlockSpec((1, P * R, C), lambda n: (n, 0, 0)),
        out_shape=jax.ShapeDtypeStruct((n_img, P * R, C), jnp.float32),
        scratch_shapes=[
            pltpu.VMEM((P * R, C), jnp.bfloat16),
            pltpu.VMEM((P * R, CM), jnp.float32),
            pltpu.VMEM((P * R, CM), jnp.bfloat16),
            pltpu.VMEM((P * R, CM), jnp.bfloat16),
            pltpu.VMEM((R, R), jnp.bfloat16),
            pltpu.SemaphoreType.DMA,
        ],
    )(proposals, pt, x0, w1t, w2t, b1p, b2p)

    y = out.reshape(n_img, 7, 7, R, C).transpose(0, 3, 4, 1, 2)
    return y.reshape(n_img * R, C, 7, 7)


# speedup vs baseline: 1.3829x; 1.1230x over previous
import jax
import jax.numpy as jnp
from jax.experimental import pallas as pl
from jax.experimental.pallas import tpu as pltpu

P = 49
R = 256
C = 256
CM = 128
TIMES = 2
NEA = 3
FAR = 3

_TAPS = [(ky - 1, kx - 1) for ky in range(3) for kx in range(3)]


def _taps():
    res = []
    for t, (dy, dx) in enumerate(_TAPS):
        y0, y1 = max(0, -dy), 7 - max(0, dy)
        x0, x1 = max(0, -dx), 7 - max(0, dx)
        spans = []
        if dx == 0:
            o = y0 * 7 * R
            spans.append((o, o + dy * 7 * R, (y1 - y0) * 7 * R))
        else:
            for yy in range(y0, y1):
                o = (yy * 7 + x0) * R
                spans.append((o, o + (dy * 7 + dx) * R, (x1 - x0) * R))
        res.append((t, spans))
    return res


_TAP_SPANS = _taps()


def _relation_kernel(p_ref, pt_ref, x0_ref, w1_ref, w2_ref, b1_ref, b2_ref,
                     out_ref, xb_ref, a1f_ref, a1t_ref, sat_ref, sa1_ref,
                     s_ref, sem):
    n = pl.program_id(0)
    cp = pltpu.make_async_copy(x0_ref.at[n], out_ref.at[0], sem)
    cp.start()
    p = p_ref[0]
    pt = pt_ref[0]
    x1c, y1c, x2c, y2c = p[:, 0:1], p[:, 1:2], p[:, 2:3], p[:, 3:4]
    x1r, y1r, x2r, y2r = pt[0:1, :], pt[1:2, :], pt[2:3, :], pt[3:4, :]
    wc = x2c - x1c + 1.0
    hc = y2c - y1c + 1.0
    wr = x2r - x1r + 1.0
    hr = y2r - y1r + 1.0
    cxc = x1c + 0.5 * wc
    cyc = y1c + 0.5 * hc
    cxr = x1r + 0.5 * wr
    cyr = y1r + 0.5 * hr
    sq = (cxc - cxr) ** 2 + (cyc - cyr) ** 2
    pos = sq > 0
    dist = jnp.where(pos, jnp.sqrt(jnp.where(pos, sq, 1.0)), 0.0)
    iw = jnp.maximum(jnp.minimum(x2r, x2c) - jnp.maximum(x1r, x1c) + 1.0, 0.0)
    ih = jnp.maximum(jnp.minimum(y2r, y2c) - jnp.maximum(y1r, y1c) + 1.0, 0.0)
    inter = iw * ih
    union = hc * wc + hr * wr - inter
    ov = inter / union
    dmax = jnp.max(dist)
    dist_nea = jnp.where(ov != 0.0, dist, 0.0)
    dist_far = jnp.where(ov == 0.0, dist, dmax)

    cols = jax.lax.broadcasted_iota(jnp.int32, (R, R), 1)
    ids = []
    ws = []
    a = dist_nea
    for _ in range(NEA):
        m = jnp.max(a, axis=1, keepdims=True)
        idx = jnp.min(jnp.where(a == m, cols, R), axis=1, keepdims=True)
        ids.append(idx)
        ws.append(m)
        a = jnp.where(cols == idx, -jnp.inf, a)
    b = dist_far
    for _ in range(FAR):
        m = jnp.min(b, axis=1, keepdims=True)
        idx = jnp.min(jnp.where(b == m, cols, R), axis=1, keepdims=True)
        ids.append(idx)
        ws.append(m)
        b = jnp.where(cols == idx, jnp.inf, b)

    w6 = jnp.concatenate(ws, axis=1) * 0.01
    mx = jnp.max(w6, axis=1, keepdims=True)
    e = jnp.exp(w6 - mx)
    w6 = e / jnp.sum(e, axis=1, keepdims=True)

    s = jnp.zeros((R, R), jnp.float32)
    for k in range(6):
        s = s + jnp.where(cols == ids[k], w6[:, k:k + 1], 0.0)
    s_ref[...] = s.astype(jnp.bfloat16)

    cp.wait()

    YR = 7 * R

    def _conv(read, rmw, w_ref):
        for dy in (-1, 0, 1):
            t = (dy + 1) * 3 + 1
            y0, y1 = max(0, -dy), 7 - max(0, dy)
            o, n = y0 * YR, (y1 - y0) * YR
            rmw(o, n, jnp.dot(read(o + dy * YR, n), w_ref[t],
                              preferred_element_type=jnp.float32))
        for dy in (-1, 0, 1):
            y0, y1 = max(0, -dy), 7 - max(0, dy)

            def body(yv, c, dy=dy):
                base = yv * YR
                for dx in (-1, 1):
                    t = (dy + 1) * 3 + (dx + 1)
                    o = base + (R if dx == -1 else 0)
                    s = o + dy * YR + dx * R
                    n = 6 * R
                    rmw(o, n, jnp.dot(read(s, n), w_ref[t],
                                      preferred_element_type=jnp.float32))
                return c

            jax.lax.fori_loop(y0, y1, body, 0)

    for _t in range(TIMES):
        xb_ref[...] = out_ref[0].astype(jnp.bfloat16)

        a1f_ref[...] = jnp.zeros((P * R, CM), jnp.float32) + b1_ref[...]

        def rmw_a1(o, n, d):
            a1f_ref[pl.ds(o, n), :] = a1f_ref[pl.ds(o, n), :] + d

        _conv(lambda o, n: xb_ref[pl.ds(o, n), :], rmw_a1, w1_ref)

        for pp in range(P):
            a1t_ref[:, pp * CM:(pp + 1) * CM] = (
                a1f_ref[pp * R:(pp + 1) * R, :].astype(jnp.bfloat16))
        sat_ref[...] = jnp.dot(
            s_ref[...], a1t_ref[...],
            preferred_element_type=jnp.float32).astype(jnp.bfloat16)
        for pp in range(P):
            sa1_ref[pp * R:(pp + 1) * R, :] = (
                sat_ref[:, pp * CM:(pp + 1) * CM])

        out_ref[0] = out_ref[0] + (jnp.zeros((P * R, C), jnp.float32)
                                   + b2_ref[...])

        def rmw_out(o, n, d):
            out_ref[0, pl.ds(o, n), :] = out_ref[0, pl.ds(o, n), :] + d

        _conv(lambda o, n: sa1_ref[pl.ds(o, n), :], rmw_out, w2_ref)


def kernel(proposals, pooled_feat, W1, b1, W2, b2):
    n_img = proposals.shape[0]
    pf = pooled_feat.reshape(n_img, R, C, 7, 7)
    x0 = pf.transpose(0, 3, 4, 1, 2).reshape(n_img, P * R, C)
    w1t = jnp.pad(W1.transpose(2, 3, 1, 0).reshape(9, C, 81),
                  ((0, 0), (0, 0), (0, CM - 81))).astype(jnp.bfloat16)
    w2t = jnp.pad(W2.transpose(2, 3, 1, 0).reshape(9, 81, C),
                  ((0, 0), (0, CM - 81), (0, 0))).astype(jnp.bfloat16)
    b1p = jnp.pad(b1, (0, CM - 81)).reshape(1, CM)
    b2p = b2.reshape(1, C)
    pt = proposals.transpose(0, 2, 1)

    out = pl.pallas_call(
        _relation_kernel,
        grid=(n_img,),
        in_specs=[
            pl.BlockSpec((1, R, 4), lambda n: (n, 0, 0)),
            pl.BlockSpec((1, 4, R), lambda n: (n, 0, 0)),
            pl.BlockSpec(memory_space=pl.ANY),
            pl.BlockSpec((9, C, CM), lambda n: (0, 0, 0)),
            pl.BlockSpec((9, CM, C), lambda n: (0, 0, 0)),
            pl.BlockSpec((1, CM), lambda n: (0, 0)),
            pl.BlockSpec((1, C), lambda n: (0, 0)),
        ],
        out_specs=pl.BlockSpec((1, P * R, C), lambda n: (n, 0, 0)),
        out_shape=jax.ShapeDtypeStruct((n_img, P * R, C), jnp.float32),
        scratch_shapes=[
            pltpu.VMEM((P * R, C), jnp.bfloat16),
            pltpu.VMEM((P * R, CM), jnp.float32),
            pltpu.VMEM((R, P * CM), jnp.bfloat16),
            pltpu.VMEM((R, P * CM), jnp.bfloat16),
            pltpu.VMEM((P * R, CM), jnp.bfloat16),
            pltpu.VMEM((R, R), jnp.bfloat16),
            pltpu.SemaphoreType.DMA,
        ],
    )(proposals, pt, x0, w1t, w2t, b1p, b2p)

    y = out.reshape(n_img, 7, 7, R, C).transpose(0, 3, 4, 1, 2)
    return y.reshape(n_img * R, C, 7, 7)
